# baseline (speedup 1.0000x reference)
import functools

import jax
import jax.image
import jax.numpy as jnp
from jax import lax
from jax.experimental import pallas as pl
from jax.experimental.pallas import tpu as pltpu
from jax.experimental.pallas import tpu_sc as plsc

_KS = 8
_B, _C, _H, _W = 4, 3, 384, 384
_N = (_H // _KS) * (_W // _KS)
_D = _C * _KS * _KS
_M2 = (_H // 2 // _KS) * (_W // 2 // _KS)
_M4 = (_H // 4 // _KS) * (_W // 4 // _KS)
_M = _N + _M2 + _M4

def _resize_mats():
    eye = jnp.eye(_H, dtype=jnp.float32)
    r2 = jax.image.resize(eye, (_H // 2, _H), method="bicubic")
    r4 = jax.image.resize(eye, (_H // 4, _H), method="bicubic")
    return r2, r4


def _patches(im):
    b, c, h, w = im.shape
    ho, wo = h // _KS, w // _KS
    im = im.reshape(b, c, ho, _KS, wo, _KS)
    im = im.transpose(0, 2, 4, 1, 3, 5)
    return im.reshape(b, ho * wo, c * _KS * _KS)


def _resize_body(g_ref, r2_ref, r2t_ref, r4_ref, r4t_ref, g2_ref, g4_ref):
    g = g_ref[0]
    t2 = jnp.dot(r2_ref[...], g, preferred_element_type=jnp.float32)
    g2_ref[0] = jnp.dot(t2, r2t_ref[...], preferred_element_type=jnp.float32)
    t4 = jnp.dot(r4_ref[...], g, preferred_element_type=jnp.float32)
    g4_ref[0] = jnp.dot(t4, r4t_ref[...], preferred_element_type=jnp.float32)


def _resize_gt(gt):
    bc = _B * _C
    r2, r4 = _resize_mats()
    g = gt.reshape(bc, _H, _W)
    g2, g4 = pl.pallas_call(
        _resize_body,
        grid=(bc,),
        in_specs=[
            pl.BlockSpec((1, _H, _W), lambda i: (i, 0, 0)),
            pl.BlockSpec((_H // 2, _H), lambda i: (0, 0)),
            pl.BlockSpec((_H, _H // 2), lambda i: (0, 0)),
            pl.BlockSpec((_H // 4, _H), lambda i: (0, 0)),
            pl.BlockSpec((_H, _H // 4), lambda i: (0, 0)),
        ],
        out_specs=[
            pl.BlockSpec((1, _H // 2, _W // 2), lambda i: (i, 0, 0)),
            pl.BlockSpec((1, _H // 4, _W // 4), lambda i: (i, 0, 0)),
        ],
        out_shape=[
            jax.ShapeDtypeStruct((bc, _H // 2, _W // 2), jnp.float32),
            jax.ShapeDtypeStruct((bc, _H // 4, _W // 4), jnp.float32),
        ],
    )(g, r2, r2.T, r4, r4.T)
    return (g2.reshape(_B, _C, _H // 2, _W // 2),
            g4.reshape(_B, _C, _H // 4, _W // 4))


_TN = 384
_NI = _N // _TN
_TM = 1008
_NJ = _M // _TM


def _score_body(c_ref, p1_ref, p2_ref, out_ref, vmin_ref, vidx_ref):
    b = pl.program_id(0)
    j = pl.program_id(2)

    @pl.when(j == 0)
    def _():
        vmin_ref[...] = jnp.full((1, _TN), jnp.inf, jnp.float32)
        vidx_ref[...] = jnp.zeros((1, _TN), jnp.int32)

    c = c_ref[0]
    q = p1_ref[0] + p2_ref[0]
    dot = lax.dot_general(c, q, (((1,), (1,)), ((), ())),
                          preferred_element_type=jnp.float32)
    cn = jnp.sum(c * c, axis=1, keepdims=True)
    s = cn - dot
    mn = jnp.min(s, axis=0, keepdims=True)
    rows = lax.broadcasted_iota(jnp.int32, (_TM, _TN), 0)
    first = jnp.min(jnp.where(s == mn, rows, _TM), axis=0, keepdims=True)
    gidx = first + (b * _M + j * _TM)
    better = mn < vmin_ref[...]
    vmin_ref[...] = jnp.where(better, mn, vmin_ref[...])
    vidx_ref[...] = jnp.where(better, gidx, vidx_ref[...])

    @pl.when(j == _NJ - 1)
    def _():
        out_ref[0] = vidx_ref[...]


def _best_buddy_idx(c, p1):
    ind = pl.pallas_call(
        _score_body,
        grid=(_B, _NI, _NJ),
        in_specs=[
            pl.BlockSpec((1, _TM, _D), lambda b, i, j: (b, j, 0)),
            pl.BlockSpec((1, _TN, _D), lambda b, i, j: (b, i, 0)),
            pl.BlockSpec((1, _TN, _D), lambda b, i, j: (b, i, 0)),
        ],
        out_specs=pl.BlockSpec((1, 1, _TN), lambda b, i, j: (b * _NI + i, 0, 0)),
        out_shape=jax.ShapeDtypeStruct((_B * _NI, 1, _TN), jnp.int32),
        scratch_shapes=[
            pltpu.VMEM((1, _TN), jnp.float32),
            pltpu.VMEM((1, _TN), jnp.int32),
        ],
        compiler_params=pltpu.CompilerParams(
            dimension_semantics=("parallel", "parallel", "arbitrary")),
    )(c, p1, c)
    return ind.reshape(_B * _N)


_NW = 32
_RPW = (_B * _N) // _NW
_CHUNK = 96
_NCH = _RPW // _CHUNK


def _sc_gather(c_flat, idx):
    mesh = plsc.VectorSubcoreMesh(core_axis_name="c", subcore_axis_name="s")

    @functools.partial(
        pl.kernel,
        mesh=mesh,
        out_type=jax.ShapeDtypeStruct((_B * _N, _D), jnp.float32),
        scratch_types=[
            pltpu.VMEM((_NCH, _CHUNK), jnp.int32),
            pltpu.VMEM((_RPW, _D), jnp.float32),
            pltpu.SemaphoreType.DMA,
        ],
        compiler_params=pltpu.CompilerParams(use_tc_tiling_on_sc=False),
    )
    def gather_kernel(c_hbm, idx_hbm, out_hbm, idx_v, rows_v, sem):
        wid = lax.axis_index("s") * 2 + lax.axis_index("c")
        base = wid * _RPW
        for t in range(_NCH):
            pltpu.sync_copy(idx_hbm.at[pl.ds(base + t * _CHUNK, _CHUNK)],
                            idx_v.at[t])
        copies = [
            pltpu.async_copy(c_hbm.at[idx_v.at[t]],
                             rows_v.at[pl.ds(t * _CHUNK, _CHUNK)], sem)
            for t in range(_NCH)
        ]
        for cp in copies:
            cp.wait()
        for t in range(_NCH):
            pltpu.sync_copy(rows_v.at[pl.ds(t * _CHUNK, _CHUNK)],
                            out_hbm.at[pl.ds(base + t * _CHUNK, _CHUNK)])

    return gather_kernel(c_flat, idx)


_RROWS = 1024
_NG = (_B * _N) // _RROWS


def _loss_body(p1_ref, sel_ref, out_ref, acc_ref):
    g = pl.program_id(0)

    @pl.when(g == 0)
    def _():
        acc_ref[0, 0] = 0.0

    acc_ref[0, 0] += jnp.sum(jnp.abs(p1_ref[...] - sel_ref[...]))

    @pl.when(g == _NG - 1)
    def _():
        out_ref[0, 0] = acc_ref[0, 0] / float(_B * _N * _D)


def _mean_l1(p1_flat, sel):
    out = pl.pallas_call(
        _loss_body,
        grid=(_NG,),
        in_specs=[
            pl.BlockSpec((_RROWS, _D), lambda g: (g, 0)),
            pl.BlockSpec((_RROWS, _D), lambda g: (g, 0)),
        ],
        out_specs=pl.BlockSpec(memory_space=pltpu.SMEM),
        out_shape=jax.ShapeDtypeStruct((1, 1), jnp.float32),
        scratch_shapes=[pltpu.SMEM((1, 1), jnp.float32)],
    )(p1_flat, sel)
    return out[0, 0]


def kernel(x, gt):
    g2, g4 = _resize_gt(gt)
    p1 = _patches(x)
    p2 = _patches(gt)
    c = jnp.concatenate([p2, _patches(g2), _patches(g4)], axis=1)
    idx = _best_buddy_idx(c, p1)
    sel = _sc_gather(c.reshape(_B * _M, _D), idx)
    return _mean_l1(p1.reshape(_B * _N, _D), sel)


# speedup vs baseline: 1.2777x; 1.2777x over previous
import functools

import jax
import jax.image
import jax.numpy as jnp
from jax import lax
from jax.experimental import pallas as pl
from jax.experimental.pallas import tpu as pltpu
from jax.experimental.pallas import tpu_sc as plsc

_KS = 8
_B, _C, _H, _W = 4, 3, 384, 384
_HO = _H // _KS
_N = _HO * _HO
_F = _KS * _KS
_M2 = (_HO // 2) * (_HO // 2)
_M4 = (_HO // 4) * (_HO // 4)
_M = _N + _M2 + _M4


def _resize_mats():
    eye = jnp.eye(_H, dtype=jnp.float32)
    r2 = jax.image.resize(eye, (_H // 2, _H), method="bicubic")
    r4 = jax.image.resize(eye, (_H // 4, _H), method="bicubic")
    return r2, r4


def _unfold(img, ho):
    return (img.reshape(ho, _KS, ho, _KS).transpose(0, 2, 1, 3)
            .reshape(ho * ho, _F))


def _prep_body(x_ref, gt_ref, r2_ref, r2t_ref, r4_ref, r4t_ref,
               p1_ref, bank_ref):
    x = x_ref[0]
    g = gt_ref[0]
    p1_ref[0] = _unfold(x, _HO)
    bank_ref[0, 0:_N] = _unfold(g, _HO)
    g2 = jnp.dot(jnp.dot(r2_ref[...], g, preferred_element_type=jnp.float32),
                 r2t_ref[...], preferred_element_type=jnp.float32)
    bank_ref[0, _N:_N + _M2] = _unfold(g2, _HO // 2)
    g4 = jnp.dot(jnp.dot(r4_ref[...], g, preferred_element_type=jnp.float32),
                 r4t_ref[...], preferred_element_type=jnp.float32)
    bank_ref[0, _N + _M2:_M] = _unfold(g4, _HO // 4)


def _prep(x, gt):
    r2, r4 = _resize_mats()
    bc = _B * _C
    p1c, bank = pl.pallas_call(
        _prep_body,
        grid=(bc,),
        in_specs=[
            pl.BlockSpec((1, _H, _W), lambda b: (b, 0, 0)),
            pl.BlockSpec((1, _H, _W), lambda b: (b, 0, 0)),
            pl.BlockSpec((_H // 2, _H), lambda b: (0, 0)),
            pl.BlockSpec((_H, _H // 2), lambda b: (0, 0)),
            pl.BlockSpec((_H // 4, _H), lambda b: (0, 0)),
            pl.BlockSpec((_H, _H // 4), lambda b: (0, 0)),
        ],
        out_specs=[
            pl.BlockSpec((1, _N, _F), lambda b: (b, 0, 0)),
            pl.BlockSpec((1, _M, _F), lambda b: (b, 0, 0)),
        ],
        out_shape=[
            jax.ShapeDtypeStruct((bc, _N, _F), jnp.float32),
            jax.ShapeDtypeStruct((bc, _M, _F), jnp.float32),
        ],
    )(x.reshape(bc, _H, _W), gt.reshape(bc, _H, _W), r2, r2.T, r4, r4.T)
    return p1c, bank


_TN = 384
_NI = _N // _TN
_TM = 1008
_NJ = _M // _TM


def _score_body(c0_ref, c1_ref, c2_ref, q20_ref, q21_ref, q22_ref,
                q10_ref, q11_ref, q12_ref, out_ref, vmin_ref, vidx_ref):
    b = pl.program_id(0)
    j = pl.program_id(2)

    @pl.when(j == 0)
    def _():
        vmin_ref[...] = jnp.full((1, _TN), jnp.inf, jnp.float32)
        vidx_ref[...] = jnp.zeros((1, _TN), jnp.int32)

    nt = (((1,), (1,)), ((), ()))
    dot = jnp.zeros((_TM, _TN), jnp.float32)
    cn = jnp.zeros((_TM, 1), jnp.float32)
    for c_ref, q1_ref, q2_ref in ((c0_ref, q10_ref, q20_ref),
                                  (c1_ref, q11_ref, q21_ref),
                                  (c2_ref, q12_ref, q22_ref)):
        c = c_ref[0, 0]
        q = q1_ref[0, 0] + q2_ref[0, 0]
        dot += lax.dot_general(c, q, nt, preferred_element_type=jnp.float32)
        cn += jnp.sum(c * c, axis=1, keepdims=True)
    s = cn - dot
    mn = jnp.min(s, axis=0, keepdims=True)
    rows = lax.broadcasted_iota(jnp.int32, (_TM, _TN), 0)
    first = jnp.min(jnp.where(s == mn, rows, _TM), axis=0, keepdims=True)
    gidx = first + (b * _C * _M + j * _TM)
    better = mn < vmin_ref[...]
    vmin_ref[...] = jnp.where(better, mn, vmin_ref[...])
    vidx_ref[...] = jnp.where(better, gidx, vidx_ref[...])

    @pl.when(j == _NJ - 1)
    def _():
        out_ref[0] = vidx_ref[...]


def _best_buddy_idx(bank4, p1c4):
    cspec = [pl.BlockSpec((1, 1, _TM, _F), lambda b, i, j, ch=ch: (b, ch, j, 0))
             for ch in range(_C)]
    qspec = [pl.BlockSpec((1, 1, _TN, _F), lambda b, i, j, ch=ch: (b, ch, i, 0))
             for ch in range(_C)]
    ind = pl.pallas_call(
        _score_body,
        grid=(_B, _NI, _NJ),
        in_specs=cspec + qspec + qspec,
        out_specs=pl.BlockSpec((1, 1, _TN), lambda b, i, j: (b * _NI + i, 0, 0)),
        out_shape=jax.ShapeDtypeStruct((_B * _NI, 1, _TN), jnp.int32),
        scratch_shapes=[
            pltpu.VMEM((1, _TN), jnp.float32),
            pltpu.VMEM((1, _TN), jnp.int32),
        ],
        compiler_params=pltpu.CompilerParams(
            dimension_semantics=("parallel", "parallel", "arbitrary")),
    )(bank4, bank4, bank4, bank4, bank4, bank4, p1c4, p1c4, p1c4)
    return ind.reshape(_B * _N)


_NW = 32
_RPW = (_B * _N) // _NW
_WPB = _N // _RPW
_CHUNK = 96
_NCH = _RPW // _CHUNK
_LANES = 16


def _sc_gather(bank_flat, idx):
    mesh = plsc.VectorSubcoreMesh(core_axis_name="c", subcore_axis_name="s")

    @functools.partial(
        pl.kernel,
        mesh=mesh,
        out_type=jax.ShapeDtypeStruct((_B * _C * _N, _F), jnp.float32),
        scratch_types=[
            pltpu.VMEM((_NCH, _CHUNK), jnp.int32),
            pltpu.VMEM((_NCH, _CHUNK), jnp.int32),
            pltpu.VMEM((_RPW, _F), jnp.float32),
            pltpu.SemaphoreType.DMA,
        ],
        compiler_params=pltpu.CompilerParams(use_tc_tiling_on_sc=False),
    )
    def gather_kernel(bank_hbm, idx_hbm, out_hbm, idx_v, idx2_v, rows_v, sem):
        wid = lax.axis_index("s") * 2 + lax.axis_index("c")
        qbase = wid * _RPW
        b = wid // _WPB
        n0 = (wid % _WPB) * _RPW
        for t in range(_NCH):
            pltpu.sync_copy(idx_hbm.at[pl.ds(qbase + t * _CHUNK, _CHUNK)],
                            idx_v.at[t])
        for ch in range(_C):
            src = idx_v if ch == 0 else idx2_v
            if ch > 0:
                for t in range(_NCH):
                    for v in range(_CHUNK // _LANES):
                        sl = pl.ds(v * _LANES, _LANES)
                        idx2_v[t, sl] = idx_v[t, sl] + ch * _M
            copies = [
                pltpu.async_copy(bank_hbm.at[src.at[t]],
                                 rows_v.at[pl.ds(t * _CHUNK, _CHUNK)], sem)
                for t in range(_NCH)
            ]
            for cp in copies:
                cp.wait()
            obase = (b * _C + ch) * _N + n0
            for t in range(_NCH):
                pltpu.sync_copy(rows_v.at[pl.ds(t * _CHUNK, _CHUNK)],
                                out_hbm.at[pl.ds(obase + t * _CHUNK, _CHUNK)])

    return gather_kernel(bank_flat, idx)


_RROWS = 1024
_NG = (_B * _C * _N) // _RROWS


def _loss_body(p1_ref, sel_ref, out_ref, acc_ref):
    g = pl.program_id(0)

    @pl.when(g == 0)
    def _():
        acc_ref[0, 0] = 0.0

    acc_ref[0, 0] += jnp.sum(jnp.abs(p1_ref[...] - sel_ref[...]))

    @pl.when(g == _NG - 1)
    def _():
        out_ref[0, 0] = acc_ref[0, 0] / float(_B * _C * _N * _F)


def _mean_l1(p1_flat, sel):
    out = pl.pallas_call(
        _loss_body,
        grid=(_NG,),
        in_specs=[
            pl.BlockSpec((_RROWS, _F), lambda g: (g, 0)),
            pl.BlockSpec((_RROWS, _F), lambda g: (g, 0)),
        ],
        out_specs=pl.BlockSpec(memory_space=pltpu.SMEM),
        out_shape=jax.ShapeDtypeStruct((1, 1), jnp.float32),
        scratch_shapes=[pltpu.SMEM((1, 1), jnp.float32)],
    )(p1_flat, sel)
    return out[0, 0]


def kernel(x, gt):
    p1c, bank = _prep(x, gt)
    idx = _best_buddy_idx(bank.reshape(_B, _C, _M, _F),
                          p1c.reshape(_B, _C, _N, _F))
    sel = _sc_gather(bank.reshape(_B * _C * _M, _F), idx)
    return _mean_l1(p1c.reshape(_B * _C * _N, _F), sel)


# speedup vs baseline: 1.3509x; 1.0573x over previous
import functools

import jax
import jax.image
import jax.numpy as jnp
from jax import lax
from jax.experimental import pallas as pl
from jax.experimental.pallas import tpu as pltpu
from jax.experimental.pallas import tpu_sc as plsc

_KS = 8
_B, _C, _H, _W = 4, 3, 384, 384
_HO = _H // _KS
_N = _HO * _HO
_F = _KS * _KS
_M2 = (_HO // 2) * (_HO // 2)
_M4 = (_HO // 4) * (_HO // 4)
_M = _N + _M2 + _M4


def _resize_mats():
    eye = jnp.eye(_H, dtype=jnp.float32)
    r2 = jax.image.resize(eye, (_H // 2, _H), method="bicubic")
    r4 = jax.image.resize(eye, (_H // 4, _H), method="bicubic")
    return r2, r4


def _unfold(img, ho):
    return (img.reshape(ho, _KS, ho, _KS).transpose(0, 2, 1, 3)
            .reshape(ho * ho, _F))


def _prep_body(x_ref, gt_ref, r2_ref, r2t_ref, r4_ref, r4t_ref,
               p1_ref, bank_ref):
    x = x_ref[0]
    g = gt_ref[0]
    p1_ref[0] = _unfold(x, _HO)
    bank_ref[0, 0:_N] = _unfold(g, _HO)
    g2 = jnp.dot(jnp.dot(r2_ref[...], g, preferred_element_type=jnp.float32),
                 r2t_ref[...], preferred_element_type=jnp.float32)
    bank_ref[0, _N:_N + _M2] = _unfold(g2, _HO // 2)
    g4 = jnp.dot(jnp.dot(r4_ref[...], g, preferred_element_type=jnp.float32),
                 r4t_ref[...], preferred_element_type=jnp.float32)
    bank_ref[0, _N + _M2:_M] = _unfold(g4, _HO // 4)


def _prep(x, gt):
    r2, r4 = _resize_mats()
    bc = _B * _C
    p1c, bank = pl.pallas_call(
        _prep_body,
        grid=(bc,),
        in_specs=[
            pl.BlockSpec((1, _H, _W), lambda b: (b, 0, 0)),
            pl.BlockSpec((1, _H, _W), lambda b: (b, 0, 0)),
            pl.BlockSpec((_H // 2, _H), lambda b: (0, 0)),
            pl.BlockSpec((_H, _H // 2), lambda b: (0, 0)),
            pl.BlockSpec((_H // 4, _H), lambda b: (0, 0)),
            pl.BlockSpec((_H, _H // 4), lambda b: (0, 0)),
        ],
        out_specs=[
            pl.BlockSpec((1, _N, _F), lambda b: (b, 0, 0)),
            pl.BlockSpec((1, _M, _F), lambda b: (b, 0, 0)),
        ],
        out_shape=[
            jax.ShapeDtypeStruct((bc, _N, _F), jnp.float32),
            jax.ShapeDtypeStruct((bc, _M, _F), jnp.float32),
        ],
    )(x.reshape(bc, _H, _W), gt.reshape(bc, _H, _W), r2, r2.T, r4, r4.T)
    return p1c, bank


_TN = 384
_NI = _N // _TN
_TM = 1008
_NJ = _M // _TM


def _score_body(c0_ref, c1_ref, c2_ref, q20_ref, q21_ref, q22_ref,
                q10_ref, q11_ref, q12_ref, out_ref, vmin_ref, vidx_ref):
    b = pl.program_id(0)
    j = pl.program_id(2)

    @pl.when(j == 0)
    def _():
        vmin_ref[...] = jnp.full((1, _TN), jnp.inf, jnp.float32)
        vidx_ref[...] = jnp.zeros((1, _TN), jnp.int32)

    nt = (((1,), (1,)), ((), ()))
    dot = jnp.zeros((_TM, _TN), jnp.float32)
    cn = jnp.zeros((_TM, 1), jnp.float32)
    for c_ref, q1_ref, q2_ref in ((c0_ref, q10_ref, q20_ref),
                                  (c1_ref, q11_ref, q21_ref),
                                  (c2_ref, q12_ref, q22_ref)):
        c = c_ref[0, 0]
        q = q1_ref[0, 0] + q2_ref[0, 0]
        dot += lax.dot_general(c, q, nt, preferred_element_type=jnp.float32)
        cn += jnp.sum(c * c, axis=1, keepdims=True)
    s = cn - dot
    mn = jnp.min(s, axis=0, keepdims=True)
    rows = lax.broadcasted_iota(jnp.int32, (_TM, _TN), 0)
    first = jnp.min(jnp.where(s == mn, rows, _TM), axis=0, keepdims=True)
    gidx = first + (b * _C * _M + j * _TM)
    better = mn < vmin_ref[...]
    vmin_ref[...] = jnp.where(better, mn, vmin_ref[...])
    vidx_ref[...] = jnp.where(better, gidx, vidx_ref[...])

    @pl.when(j == _NJ - 1)
    def _():
        out_ref[0] = vidx_ref[...]


def _best_buddy_idx(bank4, p1c4):
    cspec = [pl.BlockSpec((1, 1, _TM, _F), lambda b, i, j, ch=ch: (b, ch, j, 0))
             for ch in range(_C)]
    qspec = [pl.BlockSpec((1, 1, _TN, _F), lambda b, i, j, ch=ch: (b, ch, i, 0))
             for ch in range(_C)]
    ind = pl.pallas_call(
        _score_body,
        grid=(_B, _NI, _NJ),
        in_specs=cspec + qspec + qspec,
        out_specs=pl.BlockSpec((1, 1, _TN), lambda b, i, j: (b * _NI + i, 0, 0)),
        out_shape=jax.ShapeDtypeStruct((_B * _NI, 1, _TN), jnp.int32),
        scratch_shapes=[
            pltpu.VMEM((1, _TN), jnp.float32),
            pltpu.VMEM((1, _TN), jnp.int32),
        ],
        compiler_params=pltpu.CompilerParams(
            dimension_semantics=("parallel", "parallel", "arbitrary")),
    )(bank4, bank4, bank4, bank4, bank4, bank4, p1c4, p1c4, p1c4)
    return ind.reshape(_B * _N)


_NW = 32
_RPW = (_B * _N) // _NW
_WPB = _N // _RPW
_CHUNK = 96
_NCH = _RPW // _CHUNK
_LANES = 16


def _sc_gather_loss(bank_flat, idx, p1_flat):
    mesh = plsc.VectorSubcoreMesh(core_axis_name="c", subcore_axis_name="s")

    @functools.partial(
        pl.kernel,
        mesh=mesh,
        out_type=jax.ShapeDtypeStruct((_NW, _LANES), jnp.float32),
        scratch_types=[
            pltpu.VMEM((_NCH, _CHUNK), jnp.int32),
            pltpu.VMEM((_NCH, _CHUNK), jnp.int32),
            pltpu.VMEM((_RPW, _F), jnp.float32),
            pltpu.VMEM((_RPW, _F), jnp.float32),
            pltpu.VMEM((_LANES,), jnp.float32),
            pltpu.SemaphoreType.DMA,
        ],
        compiler_params=pltpu.CompilerParams(use_tc_tiling_on_sc=False),
    )
    def gather_kernel(bank_hbm, idx_hbm, p1_hbm, out_hbm,
                      idx_v, idx2_v, rows_v, p1_v, acc_v, sem):
        wid = lax.axis_index("s") * 2 + lax.axis_index("c")
        qbase = wid * _RPW
        b = wid // _WPB
        n0 = (wid % _WPB) * _RPW
        for t in range(_NCH):
            pltpu.sync_copy(idx_hbm.at[pl.ds(qbase + t * _CHUNK, _CHUNK)],
                            idx_v.at[t])
        acc = jnp.zeros((_LANES,), jnp.float32)
        for ch in range(_C):
            src = idx_v if ch == 0 else idx2_v
            if ch > 0:
                for t in range(_NCH):
                    for v in range(_CHUNK // _LANES):
                        sl = pl.ds(v * _LANES, _LANES)
                        idx2_v[t, sl] = idx_v[t, sl] + ch * _M
            copies = [
                pltpu.async_copy(bank_hbm.at[src.at[t]],
                                 rows_v.at[pl.ds(t * _CHUNK, _CHUNK)], sem)
                for t in range(_NCH)
            ]
            pltpu.sync_copy(p1_hbm.at[pl.ds((b * _C + ch) * _N + n0, _RPW)],
                            p1_v)
            for cp in copies:
                cp.wait()

            def row_body(r, a):
                for v in range(_F // _LANES):
                    sl = pl.ds(v * _LANES, _LANES)
                    a = a + jnp.abs(p1_v[r, sl] - rows_v[r, sl])
                return a

            acc = lax.fori_loop(0, _RPW, row_body, acc, unroll=4)
        acc_v[...] = acc
        pltpu.sync_copy(acc_v, out_hbm.at[wid])

    return gather_kernel(bank_flat, idx, p1_flat)


def kernel(x, gt):
    p1c, bank = _prep(x, gt)
    idx = _best_buddy_idx(bank.reshape(_B, _C, _M, _F),
                          p1c.reshape(_B, _C, _N, _F))
    partials = _sc_gather_loss(bank.reshape(_B * _C * _M, _F), idx,
                               p1c.reshape(_B * _C * _N, _F))
    return jnp.sum(partials) / float(_B * _C * _N * _F)
